# Initial kernel scaffold; baseline (speedup 1.0000x reference)
import jax
import jax.numpy as jnp
from jax.experimental import pallas as pl


def kernel(x):
    raise NotImplementedError("write your pallas kernel here")


# speedup vs baseline: 538.7578x; 538.7578x over previous
import jax
import jax.numpy as jnp
from jax.experimental import pallas as pl

_TB = 256


def _quant_block(x_ref, o_ref):
    xb = x_ref[0]
    x_min = jnp.min(jnp.min(xb, axis=2), axis=0)
    x_max = jnp.max(jnp.max(xb, axis=2), axis=0)
    delta = (x_max - x_min) / 255.0
    zp = jnp.round(-x_min / delta)
    d3 = delta[None, :, None]
    z3 = zp[None, :, None]
    xi = jnp.round(xb / d3) + z3
    xi = xi / 255.0
    q = jnp.clip(xi, 0.0, 1.0) * 255.0
    o_ref[0] = (q - z3) * d3


def kernel(x):
    b, h, t, c = x.shape
    return pl.pallas_call(
        _quant_block,
        grid=(b, t // _TB),
        in_specs=[pl.BlockSpec((1, h, _TB, c), lambda i, j: (i, 0, j, 0))],
        out_specs=pl.BlockSpec((1, h, _TB, c), lambda i, j: (i, 0, j, 0)),
        out_shape=jax.ShapeDtypeStruct(x.shape, x.dtype),
    )(x)


# speedup vs baseline: 590.5329x; 1.0961x over previous
import jax
import jax.numpy as jnp
from jax.experimental import pallas as pl

_TB = 256


def _quant_block(x_ref, o_ref):
    xb = x_ref[0]
    x_min = jnp.min(jnp.min(xb, axis=0), axis=1)
    x_max = jnp.max(jnp.max(xb, axis=0), axis=1)
    delta = (x_max - x_min) / 255.0
    rinv = 1.0 / delta
    zp = jnp.round(-x_min * rinv)
    r3 = rinv[None, :, None]
    z3 = zp[None, :, None]
    xi = jnp.round(xb * r3) + z3
    q = jnp.clip(xi, 0.0, 255.0)
    o_ref[0] = (q - z3) * delta[None, :, None]


def kernel(x):
    b, h, t, c = x.shape
    return pl.pallas_call(
        _quant_block,
        grid=(b, t // _TB),
        in_specs=[pl.BlockSpec((1, h, _TB, c), lambda i, j: (i, 0, j, 0))],
        out_specs=pl.BlockSpec((1, h, _TB, c), lambda i, j: (i, 0, j, 0)),
        out_shape=jax.ShapeDtypeStruct(x.shape, x.dtype),
    )(x)


# speedup vs baseline: 627.8378x; 1.0632x over previous
import jax
import jax.numpy as jnp
from jax.experimental import pallas as pl

_TB = 512


def _quant_block(x_ref, o_ref):
    xb = x_ref[0]
    x_min = jnp.min(jnp.min(xb, axis=0), axis=1)
    x_max = jnp.max(jnp.max(xb, axis=0), axis=1)
    delta = (x_max - x_min) / 255.0
    rinv = 1.0 / delta
    zp = jnp.round(-x_min * rinv)
    r3 = rinv[None, :, None]
    z3 = zp[None, :, None]
    xi = jnp.round(xb * r3) + z3
    q = jnp.clip(xi, 0.0, 255.0)
    o_ref[0] = (q - z3) * delta[None, :, None]


def kernel(x):
    b, h, t, c = x.shape
    return pl.pallas_call(
        _quant_block,
        grid=(b, t // _TB),
        in_specs=[pl.BlockSpec((1, h, _TB, c), lambda i, j: (i, 0, j, 0))],
        out_specs=pl.BlockSpec((1, h, _TB, c), lambda i, j: (i, 0, j, 0)),
        out_shape=jax.ShapeDtypeStruct(x.shape, x.dtype),
    )(x)


# speedup vs baseline: 2870.8415x; 4.5726x over previous
import jax
import jax.numpy as jnp
from jax.experimental import pallas as pl

_TB = 512


def _quant_block(x_ref, o_ref):
    xb = x_ref[0]
    x_min = jnp.min(jnp.min(xb, axis=0), axis=0)
    x_max = jnp.max(jnp.max(xb, axis=0), axis=0)
    delta = (x_max - x_min) / 255.0
    rinv = 1.0 / delta
    zp = jnp.round(-x_min * rinv)
    r3 = rinv[None, None, :]
    z3 = zp[None, None, :]
    xi = jnp.round(xb * r3) + z3
    q = jnp.clip(xi, 0.0, 255.0)
    o_ref[0] = (q - z3) * delta[None, None, :]


def kernel(x):
    b, h, t, c = x.shape
    xt = jnp.transpose(x, (0, 1, 3, 2))
    out = pl.pallas_call(
        _quant_block,
        grid=(b, t // _TB),
        in_specs=[pl.BlockSpec((1, h, c, _TB), lambda i, j: (i, 0, 0, j))],
        out_specs=pl.BlockSpec((1, h, c, _TB), lambda i, j: (i, 0, 0, j)),
        out_shape=jax.ShapeDtypeStruct((b, h, c, t), x.dtype),
    )(xt)
    return jnp.transpose(out, (0, 1, 3, 2))
